# Initial kernel scaffold; baseline (speedup 1.0000x reference)
import functools

import jax
import jax.numpy as jnp
from jax import lax
from jax.experimental import pallas as pl
from jax.experimental.pallas import tpu as pltpu
from jax.experimental.pallas import tpu_sc as plsc

N_NODES = 10000
D = 128
NUM_SC = 2
TPS = 16
TILES = NUM_SC * TPS
CH = 128
NACC = 10240
ROWS_T = NACC // TPS
DEGW = 16


def _sc_agg_body(with_deg, *refs):
    if with_deg:
        (x_hbm, src_hbm, dst_hbm, zrows_hbm, z16_hbm, ones_hbm,
         out_hbm, deg_hbm,
         srcv, dstv, rows, ones_v, zb16, acc, dacc, sem) = refs
    else:
        (x_hbm, src_hbm, dst_hbm, zrows_hbm,
         out_hbm,
         srcv, dstv, rows, acc, sem) = refs

    cid = lax.axis_index("c")
    sid = lax.axis_index("s")
    tid = cid * TPS + sid
    base = sid * ROWS_T
    nch = src_hbm.shape[0] // TILES

    pltpu.sync_copy(zrows_hbm, rows)

    @pl.loop(0, ROWS_T // CH)
    def _(k):
        pltpu.sync_copy(rows, acc.at[pl.ds(base + k * CH, CH)])

    if with_deg:
        pltpu.sync_copy(z16_hbm, zb16)
        pltpu.sync_copy(zb16, dacc.at[pl.ds(base, ROWS_T)])
        pltpu.sync_copy(ones_hbm, ones_v)

    pltpu.sync_copy(src_hbm.at[pl.ds(tid * nch, nch)], srcv)
    pltpu.sync_copy(dst_hbm.at[pl.ds(tid * nch, nch)], dstv)

    plsc.subcore_barrier()

    @pl.loop(0, nch)
    def _(j):
        pltpu.async_copy(x_hbm.at[srcv.at[j]], rows, sem).wait()
        pltpu.sync_copy(rows, acc.at[dstv.at[j]], add=True)
        if with_deg:
            pltpu.sync_copy(ones_v, dacc.at[dstv.at[j]], add=True)

    plsc.subcore_barrier()

    @pl.loop(0, ROWS_T // CH)
    def _(k):
        pltpu.sync_copy(acc.at[pl.ds(base + k * CH, CH)], rows)
        pltpu.sync_copy(rows, out_hbm.at[pl.ds(cid * NACC + base + k * CH, CH)])

    if with_deg:
        pltpu.sync_copy(dacc.at[pl.ds(base, ROWS_T)], zb16)
        pltpu.sync_copy(zb16, deg_hbm.at[pl.ds(cid * NACC + base, ROWS_T)])


def _sc_aggregate(x_pad, src_idx, dst_idx, with_deg):
    nch = src_idx.shape[0]
    mesh = plsc.VectorSubcoreMesh(core_axis_name="c", subcore_axis_name="s")
    out_type = [jax.ShapeDtypeStruct((NUM_SC * NACC, D), jnp.float32)]
    scratch = [
        pltpu.VMEM((nch // TILES, CH), jnp.int32),
        pltpu.VMEM((nch // TILES, CH), jnp.int32),
        pltpu.VMEM((CH, D), jnp.float32),
    ]
    if with_deg:
        out_type.append(jax.ShapeDtypeStruct((NUM_SC * NACC, DEGW), jnp.float32))
        scratch += [
            pltpu.VMEM((CH, DEGW), jnp.float32),
            pltpu.VMEM((ROWS_T, DEGW), jnp.float32),
        ]
    scratch.append(pltpu.VMEM_SHARED((NACC, D), jnp.float32))
    if with_deg:
        scratch.append(pltpu.VMEM_SHARED((NACC, DEGW), jnp.float32))
    scratch.append(pltpu.SemaphoreType.DMA)

    kern = pl.kernel(
        functools.partial(_sc_agg_body, with_deg),
        out_type=tuple(out_type),
        mesh=mesh,
        scratch_types=scratch,
    )
    zrows = jnp.zeros((CH, D), jnp.float32)
    if with_deg:
        z16 = jnp.zeros((ROWS_T, DEGW), jnp.float32)
        ones = jnp.ones((CH, DEGW), jnp.float32)
        return kern(x_pad, src_idx, dst_idx, zrows, z16, ones)
    return kern(x_pad, src_idx, dst_idx, zrows)


def _dense_body(relu, a_ref, d_ref, x_ref, wl_ref, wr_ref, b_ref, o_ref):
    a = a_ref[0] + a_ref[1]
    deg = d_ref[0, :, 0:1] + d_ref[1, :, 0:1]
    mean = a / jnp.maximum(deg, 1.0)
    h = jnp.dot(mean, wl_ref[...], preferred_element_type=jnp.float32)
    h = h + jnp.dot(x_ref[...], wr_ref[...], preferred_element_type=jnp.float32)
    h = h + b_ref[...]
    if relu:
        h = jnp.maximum(h, 0.0)
    o_ref[...] = h


def _dense(agg, deg, x_pad, Wl, Wr, b, relu):
    BN = 512
    grid = (NACC // BN,)
    return pl.pallas_call(
        functools.partial(_dense_body, relu),
        grid=grid,
        in_specs=[
            pl.BlockSpec((NUM_SC, BN, D), lambda i: (0, i, 0)),
            pl.BlockSpec((NUM_SC, BN, DEGW), lambda i: (0, i, 0)),
            pl.BlockSpec((BN, D), lambda i: (i, 0)),
            pl.BlockSpec((D, D), lambda i: (0, 0)),
            pl.BlockSpec((D, D), lambda i: (0, 0)),
            pl.BlockSpec((1, D), lambda i: (0, 0)),
        ],
        out_specs=pl.BlockSpec((BN, D), lambda i: (i, 0)),
        out_shape=jax.ShapeDtypeStruct((NACC, D), jnp.float32),
    )(agg, deg, x_pad, Wl, Wr, b)


def kernel(x, edge_index, Wl1, Wr1, b1, Wl2, Wr2, b2):
    E = edge_index.shape[1]
    epad = ((E + TILES * CH - 1) // (TILES * CH)) * (TILES * CH)
    src = edge_index[0].astype(jnp.int32)
    dst = edge_index[1].astype(jnp.int32)
    src_p = jnp.concatenate(
        [src, jnp.zeros((epad - E,), jnp.int32)]).reshape(-1, CH)
    dst_p = jnp.concatenate(
        [dst, jnp.full((epad - E,), N_NODES, jnp.int32)]).reshape(-1, CH)
    x_pad = jnp.concatenate(
        [x, jnp.zeros((NACC - N_NODES, D), jnp.float32)], axis=0)
    b1r = b1.reshape(1, D)
    b2r = b2.reshape(1, D)

    agg1, deg1 = _sc_aggregate(x_pad, src_p, dst_p, with_deg=True)
    agg1 = agg1.reshape(NUM_SC, NACC, D)
    deg1 = deg1.reshape(NUM_SC, NACC, DEGW)
    h = _dense(agg1, deg1, x_pad, Wl1, Wr1, b1r, relu=True)

    (agg2,) = _sc_aggregate(h, src_p, dst_p, with_deg=False)
    agg2 = agg2.reshape(NUM_SC, NACC, D)
    out = _dense(agg2, deg1, h, Wl2, Wr2, b2r, relu=False)
    return out[:N_NODES]


# speedup vs baseline: 4.0620x; 4.0620x over previous
import functools

import jax
import jax.numpy as jnp
from jax import lax
from jax.experimental import pallas as pl
from jax.experimental.pallas import tpu as pltpu
from jax.experimental.pallas import tpu_sc as plsc

N_NODES = 10000
D = 128
DH = D // 2
NUM_SC = 2
TPS = 16
CH = 128
NACC = 10240
ROWS_T = NACC // TPS
DEGW = 16


def _sc_agg_body(with_deg, *refs):
    if with_deg:
        (x_hbm, src_hbm, dst_hbm, zrows_hbm, z16_hbm, ones_hbm,
         out_hbm, deg_hbm,
         srcv, dstv, rows, ones_v, zb16, acc, dacc, sem) = refs
    else:
        (x_hbm, src_hbm, dst_hbm, zrows_hbm,
         out_hbm,
         srcv, dstv, rows, acc, sem) = refs

    cid = lax.axis_index("c")
    sid = lax.axis_index("s")
    base = sid * ROWS_T
    ncht = srcv.shape[0]
    nch_sc = src_hbm.shape[0] // NUM_SC

    pltpu.sync_copy(zrows_hbm, rows)

    @pl.loop(0, ROWS_T // CH)
    def _(k):
        pltpu.sync_copy(rows, acc.at[pl.ds(base + k * CH, CH)])

    if with_deg:
        pltpu.sync_copy(z16_hbm, zb16)
        pltpu.sync_copy(zb16, dacc.at[pl.ds(base, ROWS_T)])
        pltpu.sync_copy(ones_hbm, ones_v)

    pltpu.sync_copy(src_hbm.at[pl.ds(cid * nch_sc + sid * ncht, ncht)], srcv)
    pltpu.sync_copy(dst_hbm.at[pl.ds(sid * ncht, ncht)], dstv)

    plsc.subcore_barrier()

    @pl.loop(0, ncht)
    def _(j):
        pltpu.async_copy(x_hbm.at[srcv.at[j]], rows, sem).wait()
        pltpu.sync_copy(rows, acc.at[dstv.at[j]], add=True)

    if with_deg:
        @pl.loop(0, ncht // NUM_SC)
        def _(j):
            pltpu.sync_copy(
                ones_v, dacc.at[dstv.at[cid * (ncht // NUM_SC) + j]], add=True)

    plsc.subcore_barrier()

    @pl.loop(0, ROWS_T // CH)
    def _(k):
        pltpu.sync_copy(acc.at[pl.ds(base + k * CH, CH)], rows)
        pltpu.sync_copy(rows, out_hbm.at[pl.ds(cid * NACC + base + k * CH, CH)])

    if with_deg:
        pltpu.sync_copy(dacc.at[pl.ds(base, ROWS_T)], zb16)
        pltpu.sync_copy(zb16, deg_hbm.at[pl.ds(cid * NACC + base, ROWS_T)])


def _sc_aggregate(x_split, src_idx, dst_idx, with_deg):
    ncht = src_idx.shape[0] // NUM_SC // TPS
    mesh = plsc.VectorSubcoreMesh(core_axis_name="c", subcore_axis_name="s")
    out_type = [jax.ShapeDtypeStruct((NUM_SC * NACC, DH), jnp.float32)]
    scratch = [
        pltpu.VMEM((ncht, CH), jnp.int32),
        pltpu.VMEM((ncht, CH), jnp.int32),
        pltpu.VMEM((CH, DH), jnp.float32),
    ]
    if with_deg:
        out_type.append(jax.ShapeDtypeStruct((NUM_SC * NACC, DEGW), jnp.float32))
        scratch += [
            pltpu.VMEM((CH, DEGW), jnp.float32),
            pltpu.VMEM((ROWS_T, DEGW), jnp.float32),
        ]
    scratch.append(pltpu.VMEM_SHARED((NACC, DH), jnp.float32))
    if with_deg:
        scratch.append(pltpu.VMEM_SHARED((NACC, DEGW), jnp.float32))
    scratch.append(pltpu.SemaphoreType.DMA)

    kern = pl.kernel(
        functools.partial(_sc_agg_body, with_deg),
        out_type=tuple(out_type),
        mesh=mesh,
        scratch_types=scratch,
        compiler_params=pltpu.CompilerParams(use_tc_tiling_on_sc=False),
    )
    zrows = jnp.zeros((CH, DH), jnp.float32)
    if with_deg:
        z16 = jnp.zeros((ROWS_T, DEGW), jnp.float32)
        ones = jnp.ones((CH, DEGW), jnp.float32)
        return kern(x_split, src_idx, dst_idx, zrows, z16, ones)
    return kern(x_split, src_idx, dst_idx, zrows)


def _dense_body(relu, split_out, a_ref, d_ref, x_ref, wl_ref, wr_ref, b_ref,
                o_ref):
    a = jnp.concatenate([a_ref[0], a_ref[1]], axis=1)
    deg = d_ref[0, :, 0:1] + d_ref[1, :, 0:1]
    mean = a / jnp.maximum(deg, 1.0)
    x = jnp.concatenate([x_ref[0], x_ref[1]], axis=1)
    h = jnp.dot(mean, wl_ref[...], preferred_element_type=jnp.float32)
    h = h + jnp.dot(x, wr_ref[...], preferred_element_type=jnp.float32)
    h = h + b_ref[...]
    if relu:
        h = jnp.maximum(h, 0.0)
    if split_out:
        o_ref[0] = h[:, :DH]
        o_ref[1] = h[:, DH:]
    else:
        o_ref[...] = h


def _dense(agg, deg, x_split, Wl, Wr, b, relu, split_out):
    BN = 512
    grid = (NACC // BN,)
    if split_out:
        out_shape = jax.ShapeDtypeStruct((NUM_SC, NACC, DH), jnp.float32)
        out_spec = pl.BlockSpec((NUM_SC, BN, DH), lambda i: (0, i, 0))
    else:
        out_shape = jax.ShapeDtypeStruct((NACC, D), jnp.float32)
        out_spec = pl.BlockSpec((BN, D), lambda i: (i, 0))
    return pl.pallas_call(
        functools.partial(_dense_body, relu, split_out),
        grid=grid,
        in_specs=[
            pl.BlockSpec((NUM_SC, BN, DH), lambda i: (0, i, 0)),
            pl.BlockSpec((NUM_SC, BN, DEGW), lambda i: (0, i, 0)),
            pl.BlockSpec((NUM_SC, BN, DH), lambda i: (0, i, 0)),
            pl.BlockSpec((D, D), lambda i: (0, 0)),
            pl.BlockSpec((D, D), lambda i: (0, 0)),
            pl.BlockSpec((1, D), lambda i: (0, 0)),
        ],
        out_specs=out_spec,
        out_shape=out_shape,
    )(agg, deg, x_split, Wl, Wr, b)


def kernel(x, edge_index, Wl1, Wr1, b1, Wl2, Wr2, b2):
    E = edge_index.shape[1]
    unit = TPS * CH * 16
    epad = ((E + unit - 1) // unit) * unit
    src = edge_index[0].astype(jnp.int32)
    dst = edge_index[1].astype(jnp.int32)
    src_p = jnp.concatenate(
        [src, jnp.zeros((epad - E,), jnp.int32)]).reshape(-1, CH)
    src_p2 = jnp.concatenate([src_p, src_p + NACC], axis=0)
    dst_p = jnp.concatenate(
        [dst, jnp.full((epad - E,), N_NODES, jnp.int32)]).reshape(-1, CH)
    x_pad = jnp.concatenate(
        [x, jnp.zeros((NACC - N_NODES, D), jnp.float32)], axis=0)
    x_split = jnp.concatenate([x_pad[:, :DH], x_pad[:, DH:]], axis=0)
    b1r = b1.reshape(1, D)
    b2r = b2.reshape(1, D)

    agg1, deg1 = _sc_aggregate(x_split, src_p2, dst_p, with_deg=True)
    agg1 = agg1.reshape(NUM_SC, NACC, DH)
    deg1 = deg1.reshape(NUM_SC, NACC, DEGW)
    h_split = _dense(agg1, deg1, x_split.reshape(NUM_SC, NACC, DH),
                     Wl1, Wr1, b1r, relu=True, split_out=True)

    (agg2,) = _sc_aggregate(h_split.reshape(NUM_SC * NACC, DH),
                            src_p2, dst_p, with_deg=False)
    agg2 = agg2.reshape(NUM_SC, NACC, DH)
    out = _dense(agg2, deg1, h_split, Wl2, Wr2, b2r, relu=False,
                 split_out=False)
    return out[:N_NODES]


# speedup vs baseline: 4.7767x; 1.1759x over previous
import functools

import jax
import jax.numpy as jnp
from jax import lax
from jax.experimental import pallas as pl
from jax.experimental.pallas import tpu as pltpu
from jax.experimental.pallas import tpu_sc as plsc

N_NODES = 10000
D = 128
DH = D // 2
NUM_SC = 2
TPS = 16
CH = 128
NACC = 10240
ROWS_T = NACC // TPS
DEGW = 16


NBUF = 2


def _sc_agg_body(with_deg, *refs):
    if with_deg:
        (x_hbm, src_hbm, dst_hbm, zrows_hbm, z16_hbm, ones_hbm,
         out_hbm, deg_hbm,
         srcv, dstv, *rest) = refs
        rows = rest[0:NBUF]
        ones_v, zb16, acc, dacc = rest[NBUF:NBUF + 4]
        gsem = rest[NBUF + 4:2 * NBUF + 4]
        ssem = rest[2 * NBUF + 4:3 * NBUF + 4]
        dsem = rest[3 * NBUF + 4]
    else:
        (x_hbm, src_hbm, dst_hbm, zrows_hbm,
         out_hbm,
         srcv, dstv, *rest) = refs
        rows = rest[0:NBUF]
        acc = rest[NBUF]
        gsem = rest[NBUF + 1:2 * NBUF + 1]
        ssem = rest[2 * NBUF + 1:3 * NBUF + 1]

    cid = lax.axis_index("c")
    sid = lax.axis_index("s")
    base = sid * ROWS_T
    ncht = srcv.shape[0]
    ngrp = ncht // NBUF
    nch_sc = src_hbm.shape[0] // NUM_SC

    pltpu.sync_copy(zrows_hbm, rows[0])

    @pl.loop(0, ROWS_T // CH)
    def _(k):
        pltpu.sync_copy(rows[0], acc.at[pl.ds(base + k * CH, CH)])

    if with_deg:
        pltpu.sync_copy(z16_hbm, zb16)
        pltpu.sync_copy(zb16, dacc.at[pl.ds(base, ROWS_T)])
        pltpu.sync_copy(ones_hbm, ones_v)

    pltpu.sync_copy(src_hbm.at[pl.ds(cid * nch_sc + sid * ncht, ncht)], srcv)
    pltpu.sync_copy(dst_hbm.at[pl.ds(sid * ncht, ncht)], dstv)

    plsc.subcore_barrier()

    def _gath(j, b):
        pltpu.async_copy(x_hbm.at[srcv.at[j]], rows[b], gsem[b])

    def _scat(j, b):
        pltpu.async_copy(rows[b], acc.at[dstv.at[j]], ssem[b], add=True)

    for b in range(NBUF):
        _gath(b, b)

    @pl.loop(0, ngrp - 1)
    def _(g):
        for b in range(NBUF):
            pltpu.make_async_copy(x_hbm.at[srcv.at[g * NBUF + b]],
                                  rows[b], gsem[b]).wait()
            _scat(g * NBUF + b, b)
        for b in range(NBUF):
            pltpu.make_async_copy(rows[b], acc.at[dstv.at[g * NBUF + b]],
                                  ssem[b]).wait()
            _gath((g + 1) * NBUF + b, b)

    last = (ngrp - 1) * NBUF
    for b in range(NBUF):
        pltpu.make_async_copy(x_hbm.at[srcv.at[last + b]],
                              rows[b], gsem[b]).wait()
        _scat(last + b, b)
    for b in range(NBUF):
        pltpu.make_async_copy(rows[b], acc.at[dstv.at[last + b]],
                              ssem[b]).wait()

    if with_deg:
        half = ncht // NUM_SC
        off = cid * half

        @pl.loop(0, half // 8)
        def _(q):
            for b in range(8):
                pltpu.async_copy(
                    ones_v, dacc.at[dstv.at[off + q * 8 + b]], dsem, add=True)
            for b in range(8):
                pltpu.make_async_copy(
                    ones_v, dacc.at[dstv.at[off + q * 8 + b]], dsem).wait()

    plsc.subcore_barrier()

    @pl.loop(0, ROWS_T // CH)
    def _(k):
        pltpu.sync_copy(acc.at[pl.ds(base + k * CH, CH)], rows[0])
        pltpu.sync_copy(rows[0],
                        out_hbm.at[pl.ds(cid * NACC + base + k * CH, CH)])

    if with_deg:
        pltpu.sync_copy(dacc.at[pl.ds(base, ROWS_T)], zb16)
        pltpu.sync_copy(zb16, deg_hbm.at[pl.ds(cid * NACC + base, ROWS_T)])


def _sc_aggregate(x_split, src_idx, dst_idx, with_deg):
    ncht = src_idx.shape[0] // NUM_SC // TPS
    mesh = plsc.VectorSubcoreMesh(core_axis_name="c", subcore_axis_name="s")
    out_type = [jax.ShapeDtypeStruct((NUM_SC * NACC, DH), jnp.float32)]
    scratch = [
        pltpu.VMEM((ncht, CH), jnp.int32),
        pltpu.VMEM((ncht, CH), jnp.int32),
    ]
    scratch += [pltpu.VMEM((CH, DH), jnp.float32) for _ in range(NBUF)]
    if with_deg:
        out_type.append(jax.ShapeDtypeStruct((NUM_SC * NACC, DEGW), jnp.float32))
        scratch += [
            pltpu.VMEM((CH, DEGW), jnp.float32),
            pltpu.VMEM((ROWS_T, DEGW), jnp.float32),
        ]
    scratch.append(pltpu.VMEM_SHARED((NACC, DH), jnp.float32))
    if with_deg:
        scratch.append(pltpu.VMEM_SHARED((NACC, DEGW), jnp.float32))
    scratch += [pltpu.SemaphoreType.DMA for _ in range(2 * NBUF)]
    if with_deg:
        scratch.append(pltpu.SemaphoreType.DMA)

    kern = pl.kernel(
        functools.partial(_sc_agg_body, with_deg),
        out_type=tuple(out_type),
        mesh=mesh,
        scratch_types=scratch,
        compiler_params=pltpu.CompilerParams(use_tc_tiling_on_sc=False),
    )
    zrows = jnp.zeros((CH, DH), jnp.float32)
    if with_deg:
        z16 = jnp.zeros((ROWS_T, DEGW), jnp.float32)
        ones = jnp.ones((CH, DEGW), jnp.float32)
        return kern(x_split, src_idx, dst_idx, zrows, z16, ones)
    return kern(x_split, src_idx, dst_idx, zrows)


def _dense_body(relu, split_out, a_ref, d_ref, x_ref, wl_ref, wr_ref, b_ref,
                o_ref):
    a = jnp.concatenate([a_ref[0], a_ref[1]], axis=1)
    deg = d_ref[0, :, 0:1] + d_ref[1, :, 0:1]
    mean = a / jnp.maximum(deg, 1.0)
    x = jnp.concatenate([x_ref[0], x_ref[1]], axis=1)
    h = jnp.dot(mean, wl_ref[...], preferred_element_type=jnp.float32)
    h = h + jnp.dot(x, wr_ref[...], preferred_element_type=jnp.float32)
    h = h + b_ref[...]
    if relu:
        h = jnp.maximum(h, 0.0)
    if split_out:
        o_ref[0] = h[:, :DH]
        o_ref[1] = h[:, DH:]
    else:
        o_ref[...] = h


def _dense(agg, deg, x_split, Wl, Wr, b, relu, split_out):
    BN = 512
    grid = (NACC // BN,)
    if split_out:
        out_shape = jax.ShapeDtypeStruct((NUM_SC, NACC, DH), jnp.float32)
        out_spec = pl.BlockSpec((NUM_SC, BN, DH), lambda i: (0, i, 0))
    else:
        out_shape = jax.ShapeDtypeStruct((NACC, D), jnp.float32)
        out_spec = pl.BlockSpec((BN, D), lambda i: (i, 0))
    return pl.pallas_call(
        functools.partial(_dense_body, relu, split_out),
        grid=grid,
        in_specs=[
            pl.BlockSpec((NUM_SC, BN, DH), lambda i: (0, i, 0)),
            pl.BlockSpec((NUM_SC, BN, DEGW), lambda i: (0, i, 0)),
            pl.BlockSpec((NUM_SC, BN, DH), lambda i: (0, i, 0)),
            pl.BlockSpec((D, D), lambda i: (0, 0)),
            pl.BlockSpec((D, D), lambda i: (0, 0)),
            pl.BlockSpec((1, D), lambda i: (0, 0)),
        ],
        out_specs=out_spec,
        out_shape=out_shape,
    )(agg, deg, x_split, Wl, Wr, b)


def kernel(x, edge_index, Wl1, Wr1, b1, Wl2, Wr2, b2):
    E = edge_index.shape[1]
    unit = TPS * CH * 16
    epad = ((E + unit - 1) // unit) * unit
    src = edge_index[0].astype(jnp.int32)
    dst = edge_index[1].astype(jnp.int32)
    src_p = jnp.concatenate(
        [src, jnp.zeros((epad - E,), jnp.int32)]).reshape(-1, CH)
    src_p2 = jnp.concatenate([src_p, src_p + NACC], axis=0)
    dst_p = jnp.concatenate(
        [dst, jnp.full((epad - E,), N_NODES, jnp.int32)]).reshape(-1, CH)
    x_pad = jnp.concatenate(
        [x, jnp.zeros((NACC - N_NODES, D), jnp.float32)], axis=0)
    x_split = jnp.concatenate([x_pad[:, :DH], x_pad[:, DH:]], axis=0)
    b1r = b1.reshape(1, D)
    b2r = b2.reshape(1, D)

    agg1, deg1 = _sc_aggregate(x_split, src_p2, dst_p, with_deg=True)
    agg1 = agg1.reshape(NUM_SC, NACC, DH)
    deg1 = deg1.reshape(NUM_SC, NACC, DEGW)
    h_split = _dense(agg1, deg1, x_split.reshape(NUM_SC, NACC, DH),
                     Wl1, Wr1, b1r, relu=True, split_out=True)

    (agg2,) = _sc_aggregate(h_split.reshape(NUM_SC * NACC, DH),
                            src_p2, dst_p, with_deg=False)
    agg2 = agg2.reshape(NUM_SC, NACC, DH)
    out = _dense(agg2, deg1, h_split, Wl2, Wr2, b2r, relu=False,
                 split_out=False)
    return out[:N_NODES]


# speedup vs baseline: 5.1539x; 1.0790x over previous
import functools

import jax
import jax.numpy as jnp
from jax import lax
from jax.experimental import pallas as pl
from jax.experimental.pallas import tpu as pltpu
from jax.experimental.pallas import tpu_sc as plsc

N_NODES = 10000
D = 128
DH = D // 2
NUM_SC = 2
TPS = 16
CH = 128
NACC = 10240
ROWS_T = NACC // TPS
DEGW = 16


NBUF = 4
NPHASE = 2


def _sc_agg_body(with_deg, *refs):
    if with_deg:
        (x_hbm, src_hbm, dst_hbm, zrows_hbm, z16_hbm, ones_hbm,
         out_hbm, deg_hbm,
         srcv, dstv, *rest) = refs
        rows = rest[0:NBUF]
        ones_v, acc, dacc = rest[NBUF:NBUF + 3]
        gsem = rest[NBUF + 3:2 * NBUF + 3]
        ssem = rest[2 * NBUF + 3:3 * NBUF + 3]
        dsem = rest[3 * NBUF + 3]
    else:
        (x_hbm, src_hbm, dst_hbm, zrows_hbm,
         out_hbm,
         srcv, dstv, *rest) = refs
        rows = rest[0:NBUF]
        acc = rest[NBUF]
        gsem = rest[NBUF + 1:2 * NBUF + 1]
        ssem = rest[2 * NBUF + 1:3 * NBUF + 1]

    cid = lax.axis_index("c")
    sid = lax.axis_index("s")
    base = sid * ROWS_T
    nchp = srcv.shape[0]
    ncht = nchp * NPHASE
    ngrp = nchp // NBUF
    nch_sc = src_hbm.shape[0] // NUM_SC

    pltpu.sync_copy(zrows_hbm, rows[0])

    @pl.loop(0, ROWS_T // CH)
    def _(k):
        pltpu.sync_copy(rows[0], acc.at[pl.ds(base + k * CH, CH)])

    if with_deg:
        pltpu.sync_copy(z16_hbm, ones_v)

        @pl.loop(0, ROWS_T // CH)
        def _(k):
            pltpu.sync_copy(ones_v, dacc.at[pl.ds(base + k * CH, CH)])
        pltpu.sync_copy(ones_hbm, ones_v)

    plsc.subcore_barrier()

    def _gath(j, b):
        pltpu.async_copy(x_hbm.at[srcv.at[j]], rows[b], gsem[b])

    def _scat(j, b):
        pltpu.async_copy(rows[b], acc.at[dstv.at[j]], ssem[b], add=True)

    for p in range(NPHASE):
        off = sid * ncht + p * nchp
        pltpu.sync_copy(src_hbm.at[pl.ds(cid * nch_sc + off, nchp)], srcv)
        pltpu.sync_copy(dst_hbm.at[pl.ds(off, nchp)], dstv)

        for b in range(NBUF):
            _gath(b, b)

        @pl.loop(0, ngrp - 1)
        def _(g):
            for b in range(NBUF):
                pltpu.make_async_copy(x_hbm.at[srcv.at[g * NBUF + b]],
                                      rows[b], gsem[b]).wait()
                _scat(g * NBUF + b, b)
            for b in range(NBUF):
                pltpu.make_async_copy(rows[b], acc.at[dstv.at[g * NBUF + b]],
                                      ssem[b]).wait()
                _gath((g + 1) * NBUF + b, b)

        last = (ngrp - 1) * NBUF
        for b in range(NBUF):
            pltpu.make_async_copy(x_hbm.at[srcv.at[last + b]],
                                  rows[b], gsem[b]).wait()
            _scat(last + b, b)
        for b in range(NBUF):
            pltpu.make_async_copy(rows[b], acc.at[dstv.at[last + b]],
                                  ssem[b]).wait()

        if with_deg:
            @pl.when(cid == p)
            def _():
                @pl.loop(0, nchp // 8)
                def _(q):
                    for b in range(8):
                        pltpu.async_copy(
                            ones_v, dacc.at[dstv.at[q * 8 + b]], dsem,
                            add=True)
                    for b in range(8):
                        pltpu.make_async_copy(
                            ones_v, dacc.at[dstv.at[q * 8 + b]], dsem).wait()

    plsc.subcore_barrier()

    @pl.loop(0, ROWS_T // CH)
    def _(k):
        pltpu.sync_copy(acc.at[pl.ds(base + k * CH, CH)], rows[0])
        pltpu.sync_copy(rows[0],
                        out_hbm.at[pl.ds(cid * NACC + base + k * CH, CH)])

    if with_deg:
        @pl.loop(0, ROWS_T // CH)
        def _(k):
            pltpu.sync_copy(dacc.at[pl.ds(base + k * CH, CH)], ones_v)
            pltpu.sync_copy(
                ones_v, deg_hbm.at[pl.ds(cid * NACC + base + k * CH, CH)])


def _sc_aggregate(x_split, src_idx, dst_idx, with_deg):
    nchp = src_idx.shape[0] // NUM_SC // TPS // NPHASE
    mesh = plsc.VectorSubcoreMesh(core_axis_name="c", subcore_axis_name="s")
    out_type = [jax.ShapeDtypeStruct((NUM_SC * NACC, DH), jnp.float32)]
    scratch = [
        pltpu.VMEM((nchp, CH), jnp.int32),
        pltpu.VMEM((nchp, CH), jnp.int32),
    ]
    scratch += [pltpu.VMEM((CH, DH), jnp.float32) for _ in range(NBUF)]
    if with_deg:
        out_type.append(jax.ShapeDtypeStruct((NUM_SC * NACC, DEGW), jnp.float32))
        scratch.append(pltpu.VMEM((CH, DEGW), jnp.float32))
    scratch.append(pltpu.VMEM_SHARED((NACC, DH), jnp.float32))
    if with_deg:
        scratch.append(pltpu.VMEM_SHARED((NACC, DEGW), jnp.float32))
    scratch += [pltpu.SemaphoreType.DMA for _ in range(2 * NBUF)]
    if with_deg:
        scratch.append(pltpu.SemaphoreType.DMA)

    kern = pl.kernel(
        functools.partial(_sc_agg_body, with_deg),
        out_type=tuple(out_type),
        mesh=mesh,
        scratch_types=scratch,
        compiler_params=pltpu.CompilerParams(use_tc_tiling_on_sc=False),
    )
    zrows = jnp.zeros((CH, DH), jnp.float32)
    if with_deg:
        z16 = jnp.zeros((CH, DEGW), jnp.float32)
        ones = jnp.ones((CH, DEGW), jnp.float32)
        return kern(x_split, src_idx, dst_idx, zrows, z16, ones)
    return kern(x_split, src_idx, dst_idx, zrows)


def _dense_body(relu, split_out, a_ref, d_ref, x_ref, wl_ref, wr_ref, b_ref,
                o_ref):
    a = jnp.concatenate([a_ref[0], a_ref[1]], axis=1)
    deg = d_ref[0, :, 0:1] + d_ref[1, :, 0:1]
    mean = a / jnp.maximum(deg, 1.0)
    x = jnp.concatenate([x_ref[0], x_ref[1]], axis=1)
    h = jnp.dot(mean, wl_ref[...], preferred_element_type=jnp.float32)
    h = h + jnp.dot(x, wr_ref[...], preferred_element_type=jnp.float32)
    h = h + b_ref[...]
    if relu:
        h = jnp.maximum(h, 0.0)
    if split_out:
        o_ref[0] = h[:, :DH]
        o_ref[1] = h[:, DH:]
    else:
        o_ref[...] = h


def _dense(agg, deg, x_split, Wl, Wr, b, relu, split_out):
    BN = 512
    grid = (NACC // BN,)
    if split_out:
        out_shape = jax.ShapeDtypeStruct((NUM_SC, NACC, DH), jnp.float32)
        out_spec = pl.BlockSpec((NUM_SC, BN, DH), lambda i: (0, i, 0))
    else:
        out_shape = jax.ShapeDtypeStruct((NACC, D), jnp.float32)
        out_spec = pl.BlockSpec((BN, D), lambda i: (i, 0))
    return pl.pallas_call(
        functools.partial(_dense_body, relu, split_out),
        grid=grid,
        in_specs=[
            pl.BlockSpec((NUM_SC, BN, DH), lambda i: (0, i, 0)),
            pl.BlockSpec((NUM_SC, BN, DEGW), lambda i: (0, i, 0)),
            pl.BlockSpec((NUM_SC, BN, DH), lambda i: (0, i, 0)),
            pl.BlockSpec((D, D), lambda i: (0, 0)),
            pl.BlockSpec((D, D), lambda i: (0, 0)),
            pl.BlockSpec((1, D), lambda i: (0, 0)),
        ],
        out_specs=out_spec,
        out_shape=out_shape,
    )(agg, deg, x_split, Wl, Wr, b)


def kernel(x, edge_index, Wl1, Wr1, b1, Wl2, Wr2, b2):
    E = edge_index.shape[1]
    unit = TPS * CH * 16
    epad = ((E + unit - 1) // unit) * unit
    src = edge_index[0].astype(jnp.int32)
    dst = edge_index[1].astype(jnp.int32)
    src_p = jnp.concatenate(
        [src, jnp.zeros((epad - E,), jnp.int32)]).reshape(-1, CH)
    src_p2 = jnp.concatenate([src_p, src_p + NACC], axis=0)
    dst_p = jnp.concatenate(
        [dst, jnp.full((epad - E,), N_NODES, jnp.int32)]).reshape(-1, CH)
    x_pad = jnp.concatenate(
        [x, jnp.zeros((NACC - N_NODES, D), jnp.float32)], axis=0)
    x_split = jnp.concatenate([x_pad[:, :DH], x_pad[:, DH:]], axis=0)
    b1r = b1.reshape(1, D)
    b2r = b2.reshape(1, D)

    agg1, deg1 = _sc_aggregate(x_split, src_p2, dst_p, with_deg=True)
    agg1 = agg1.reshape(NUM_SC, NACC, DH)
    deg1 = deg1.reshape(NUM_SC, NACC, DEGW)
    h_split = _dense(agg1, deg1, x_split.reshape(NUM_SC, NACC, DH),
                     Wl1, Wr1, b1r, relu=True, split_out=True)

    (agg2,) = _sc_aggregate(h_split.reshape(NUM_SC * NACC, DH),
                            src_p2, dst_p, with_deg=False)
    agg2 = agg2.reshape(NUM_SC, NACC, DH)
    out = _dense(agg2, deg1, h_split, Wl2, Wr2, b2r, relu=False,
                 split_out=False)
    return out[:N_NODES]
